# baseline (speedup 1.0000x reference)
import functools
import jax
import jax.numpy as jnp
from jax import lax
from jax.experimental import pallas as pl
from jax.experimental.pallas import tpu as pltpu
from jax.experimental.pallas import tpu_sc as plsc

N_USERS = 4096
NUM_SUBREDDITS = 20000
X_DIM = 64
SUB_REP_DIM = 3

NC = 2
NS = 16
NW = NC * NS
L = 16
UPT = N_USERS // NW
NV = UPT // L
CK = 160
NCHUNK = NUM_SUBREDDITS // CK
RE = SUB_REP_DIM * L


def _make_sc_matmul():
    mesh = plsc.VectorSubcoreMesh(core_axis_name="c", subcore_axis_name="s")

    @functools.partial(
        pl.kernel,
        mesh=mesh,
        out_type=jax.ShapeDtypeStruct((SUB_REP_DIM, N_USERS), jnp.float32),
        scratch_types=[
            pltpu.VMEM((2, CK, UPT), jnp.float32),
            pltpu.VMEM((2, CK, RE), jnp.float32),
            pltpu.VMEM((SUB_REP_DIM, UPT), jnp.float32),
            pltpu.SemaphoreType.DMA((2, 2)),
        ],
    )
    def sc_matmul(st_hbm, re_hbm, out_hbm, sbuf, rbuf, stage, sems):
        wid = lax.axis_index("s") * NC + lax.axis_index("c")
        c0 = wid * UPT

        def start(g, slot):
            pltpu.make_async_copy(
                st_hbm.at[pl.ds(g * CK, CK), pl.ds(c0, UPT)],
                sbuf.at[slot],
                sems.at[slot, 0],
            ).start()
            pltpu.make_async_copy(
                re_hbm.at[pl.ds(g * CK, CK), :],
                rbuf.at[slot],
                sems.at[slot, 1],
            ).start()

        def wait(g, slot):
            pltpu.make_async_copy(
                st_hbm.at[pl.ds(g * CK, CK), pl.ds(c0, UPT)],
                sbuf.at[slot],
                sems.at[slot, 0],
            ).wait()
            pltpu.make_async_copy(
                re_hbm.at[pl.ds(g * CK, CK), :],
                rbuf.at[slot],
                sems.at[slot, 1],
            ).wait()

        start(0, 0)
        start(1, 1)

        def chunk_body(g, accs):
            slot = lax.rem(g, 2)
            wait(g, slot)

            def k_body(k, accs):
                out = []
                for j in range(SUB_REP_DIM):
                    r_j = rbuf[slot, k, pl.ds(j * L, L)]
                    for v in range(NV):
                        s_v = sbuf[slot, k, pl.ds(v * L, L)]
                        out.append(accs[j * NV + v] + s_v * r_j)
                return tuple(out)

            accs = lax.fori_loop(0, CK, k_body, accs)

            @pl.when(g + 2 < NCHUNK)
            def _():
                start(g + 2, slot)

            return accs

        zero = jnp.zeros((L,), jnp.float32)
        accs = lax.fori_loop(
            0, NCHUNK, chunk_body, tuple([zero] * (SUB_REP_DIM * NV)))

        for j in range(SUB_REP_DIM):
            for v in range(NV):
                stage[j, pl.ds(v * L, L)] = accs[j * NV + v]
        pltpu.sync_copy(stage, out_hbm.at[:, pl.ds(c0, UPT)])

    return sc_matmul


_sc_matmul = _make_sc_matmul()


def kernel(x, S, R):
    ST = S.T
    re_exp = jnp.broadcast_to(
        R[:, :, None], (NUM_SUBREDDITS, SUB_REP_DIM, L)
    ).reshape(NUM_SUBREDDITS, RE)
    o_t = _sc_matmul(ST, re_exp)
    return jnp.concatenate((x, o_t.T), axis=1)


# speedup vs baseline: 1.8788x; 1.8788x over previous
import functools
import jax
import jax.numpy as jnp
from jax import lax
from jax.experimental import pallas as pl
from jax.experimental.pallas import tpu as pltpu
from jax.experimental.pallas import tpu_sc as plsc

N_USERS = 4096
NUM_SUBREDDITS = 20000
X_DIM = 64
SUB_REP_DIM = 3

K_TC = 15360
K_SC = NUM_SUBREDDITS - K_TC

BK = 512
NSTEPS = 30
KMAIN = NSTEPS * BK


def _tc_kernel(st_ref, rt_ref, o_ref):
    i = pl.program_id(0)
    st = st_ref[...].astype(jnp.bfloat16)
    rt = rt_ref[:, pl.ds(i * BK, BK)].astype(jnp.bfloat16)
    acc = lax.dot_general(
        rt, st,
        dimension_numbers=(((1,), (0,)), ((), ())),
        preferred_element_type=jnp.float32,
    )

    @pl.when(i == 0)
    def _():
        o_ref[...] = acc

    @pl.when(i != 0)
    def _():
        o_ref[...] = o_ref[...] + acc


def _tc_matmul(ST, RT):
    return pl.pallas_call(
        _tc_kernel,
        grid=(NSTEPS,),
        in_specs=[
            pl.BlockSpec((BK, N_USERS), lambda i: (i, 0)),
            pl.BlockSpec((SUB_REP_DIM, NUM_SUBREDDITS), lambda i: (0, 0)),
        ],
        out_specs=pl.BlockSpec((SUB_REP_DIM, N_USERS), lambda i: (0, 0)),
        out_shape=jax.ShapeDtypeStruct((SUB_REP_DIM, N_USERS), jnp.float32),
        compiler_params=pltpu.CompilerParams(
            dimension_semantics=("arbitrary",),
            vmem_limit_bytes=100 * 1024 * 1024,
        ),
    )(ST, RT)


NC = 2
NS = 16
NW = NC * NS
L = 16
UPT = N_USERS // NW
NV = UPT // L
CK = 160
NCHUNK = K_SC // CK
RE = SUB_REP_DIM * L


def _make_sc_matmul():
    mesh = plsc.VectorSubcoreMesh(core_axis_name="c", subcore_axis_name="s")

    @functools.partial(
        pl.kernel,
        mesh=mesh,
        out_type=jax.ShapeDtypeStruct((SUB_REP_DIM, N_USERS), jnp.float32),
        scratch_types=[
            pltpu.VMEM((2, CK, UPT), jnp.float32),
            pltpu.VMEM((2, CK, RE), jnp.float32),
            pltpu.VMEM((SUB_REP_DIM, UPT), jnp.float32),
            pltpu.SemaphoreType.DMA((2, 2)),
        ],
    )
    def sc_matmul(st_hbm, re_hbm, out_hbm, sbuf, rbuf, stage, sems):
        wid = lax.axis_index("s") * NC + lax.axis_index("c")
        c0 = wid * UPT

        def start(g, slot):
            pltpu.make_async_copy(
                st_hbm.at[pl.ds(K_TC + g * CK, CK), pl.ds(c0, UPT)],
                sbuf.at[slot],
                sems.at[slot, 0],
            ).start()
            pltpu.make_async_copy(
                re_hbm.at[pl.ds(K_TC + g * CK, CK), :],
                rbuf.at[slot],
                sems.at[slot, 1],
            ).start()

        def wait(g, slot):
            pltpu.make_async_copy(
                st_hbm.at[pl.ds(K_TC + g * CK, CK), pl.ds(c0, UPT)],
                sbuf.at[slot],
                sems.at[slot, 0],
            ).wait()
            pltpu.make_async_copy(
                re_hbm.at[pl.ds(K_TC + g * CK, CK), :],
                rbuf.at[slot],
                sems.at[slot, 1],
            ).wait()

        start(0, 0)
        start(1, 1)

        def chunk_body(g, accs):
            slot = lax.rem(g, 2)
            wait(g, slot)

            def k_body(k, accs):
                out = []
                for j in range(SUB_REP_DIM):
                    r_j = rbuf[slot, k, pl.ds(j * L, L)]
                    for v in range(NV):
                        s_v = sbuf[slot, k, pl.ds(v * L, L)]
                        out.append(accs[j * NV + v] + s_v * r_j)
                return tuple(out)

            accs = lax.fori_loop(0, CK, k_body, accs)

            @pl.when(g + 2 < NCHUNK)
            def _():
                start(g + 2, slot)

            return accs

        zero = jnp.zeros((L,), jnp.float32)
        accs = lax.fori_loop(
            0, NCHUNK, chunk_body, tuple([zero] * (SUB_REP_DIM * NV)))

        for j in range(SUB_REP_DIM):
            for v in range(NV):
                stage[j, pl.ds(v * L, L)] = accs[j * NV + v]
        pltpu.sync_copy(stage, out_hbm.at[:, pl.ds(c0, UPT)])

    return sc_matmul


_sc_matmul = _make_sc_matmul()


def kernel(x, S, R):
    ST = S.T
    RT = R.T
    re_exp = jnp.broadcast_to(
        R[:, :, None], (NUM_SUBREDDITS, SUB_REP_DIM, L)
    ).reshape(NUM_SUBREDDITS, RE)
    o_t_sc = _sc_matmul(ST, re_exp)
    o_t_tc = _tc_matmul(ST, RT)
    o_t = o_t_tc + o_t_sc
    return jnp.concatenate((x, o_t.T), axis=1)


# speedup vs baseline: 3.2156x; 1.7115x over previous
import jax
import jax.numpy as jnp
from jax import lax
from jax.experimental import pallas as pl
from jax.experimental.pallas import tpu as pltpu

N_USERS = 4096
NUM_SUBREDDITS = 20000
X_DIM = 64
SUB_REP_DIM = 3

BK = 512
NSTEPS = 39
KMAIN = NSTEPS * BK
KREM = NUM_SUBREDDITS - KMAIN


def _agg_kernel(st_ref, strem_ref, rt_ref, o_ref):
    i = pl.program_id(0)
    st = st_ref[...].astype(jnp.bfloat16)
    rt = rt_ref[:, pl.ds(i * BK, BK)].astype(jnp.bfloat16)
    acc = lax.dot_general(
        rt, st,
        dimension_numbers=(((1,), (0,)), ((), ())),
        preferred_element_type=jnp.float32,
    )

    @pl.when(i == 0)
    def _():
        rem = lax.dot_general(
            rt_ref[:, KMAIN:].astype(jnp.bfloat16),
            strem_ref[...].astype(jnp.bfloat16),
            dimension_numbers=(((1,), (0,)), ((), ())),
            preferred_element_type=jnp.float32,
        )
        o_ref[...] = acc + rem

    @pl.when(i != 0)
    def _():
        o_ref[...] = o_ref[...] + acc


def kernel(x, S, R):
    ST = S.T
    RT = R.T
    o_t = pl.pallas_call(
        _agg_kernel,
        grid=(NSTEPS,),
        in_specs=[
            pl.BlockSpec((BK, N_USERS), lambda i: (i, 0)),
            pl.BlockSpec((KREM, N_USERS), lambda i: (NSTEPS * (BK // KREM), 0)),
            pl.BlockSpec((SUB_REP_DIM, NUM_SUBREDDITS), lambda i: (0, 0)),
        ],
        out_specs=pl.BlockSpec((SUB_REP_DIM, N_USERS), lambda i: (0, 0)),
        out_shape=jax.ShapeDtypeStruct((SUB_REP_DIM, N_USERS), jnp.float32),
        compiler_params=pltpu.CompilerParams(
            dimension_semantics=("arbitrary",),
            vmem_limit_bytes=100 * 1024 * 1024,
        ),
    )(ST, ST, RT)
    return jnp.concatenate((x, o_t.T), axis=1)
